# baseline (speedup 1.0000x reference)
import jax
import jax.numpy as jnp
from jax.experimental import pallas as pl
from jax.experimental.pallas import tpu as pltpu

V = 10000
C = 128
O = 128
NMID = 24
MID = 400
TAIL = 200
NSTEPS = NMID + 2


def _in_copy(adj_hbm, abuf, isems, start, rows, slot):
    return pltpu.make_async_copy(
        adj_hbm.at[pl.ds(start, rows), :],
        abuf.at[slot, pl.ds(0, rows), :],
        isems.at[slot],
    )


def _out_copy(obuf, out_hbm, osems, start, rows, slot):
    return pltpu.make_async_copy(
        obuf.at[slot, pl.ds(0, rows), :],
        out_hbm.at[pl.ds(start, rows), :],
        osems.at[slot],
    )


def _dot_bf16(a, h):
    return jax.lax.dot_general(
        a, h,
        dimension_numbers=(((1,), (0,)), ((), ())),
        precision=jax.lax.Precision('bfloat16'),
        preferred_element_type=jnp.float32,
    )


def _manual_kernel(x_ref, w_ref, b_ref, adj_hbm, out_hbm,
                   h_ref, abuf, obuf, isems, osems):
    i = pl.program_id(0)
    slot = jax.lax.rem(i, 2)
    nslot = jax.lax.rem(i + 1, 2)

    @pl.when(i == 0)
    def _():
        _in_copy(adj_hbm, abuf, isems, 0, MID, 0).start()
        h = jax.lax.dot_general(
            x_ref[...], w_ref[...],
            dimension_numbers=(((1,), (1,)), ((), ())),
            preferred_element_type=jnp.float32,
        )
        h_ref[...] = h + b_ref[...]

    @pl.when(i <= NMID - 2)
    def _():
        _in_copy(adj_hbm, abuf, isems, MID * (i + 1), MID, nslot).start()

    @pl.when(i == NMID - 1)
    def _():
        _in_copy(adj_hbm, abuf, isems, NMID * MID, TAIL, nslot).start()

    @pl.when(i == NMID)
    def _():
        _in_copy(adj_hbm, abuf, isems, NMID * MID + TAIL, TAIL, nslot).start()

    @pl.when(i <= NMID - 1)
    def _():
        _in_copy(adj_hbm, abuf, isems, MID * i, MID, slot).wait()
        p = _dot_bf16(abuf[slot, 0:MID, :], h_ref[...])

        @pl.when(i >= 2)
        def _():
            _out_copy(obuf, out_hbm, osems, MID * (i - 2), MID, slot).wait()

        obuf[slot, 0:MID, :] = p
        _out_copy(obuf, out_hbm, osems, MID * i, MID, slot).start()

    @pl.when(i == NMID)
    def _():
        start = NMID * MID
        _in_copy(adj_hbm, abuf, isems, start, TAIL, slot).wait()
        p = _dot_bf16(abuf[slot, 0:TAIL, :], h_ref[...])
        _out_copy(obuf, out_hbm, osems, MID * (i - 2), MID, slot).wait()
        obuf[slot, 0:TAIL, :] = p
        _out_copy(obuf, out_hbm, osems, start, TAIL, slot).start()

    @pl.when(i == NMID + 1)
    def _():
        start = NMID * MID + TAIL
        _in_copy(adj_hbm, abuf, isems, start, TAIL, slot).wait()
        p = _dot_bf16(abuf[slot, 0:TAIL, :], h_ref[...])
        _out_copy(obuf, out_hbm, osems, MID * (i - 2), MID, slot).wait()
        obuf[slot, 0:TAIL, :] = p
        _out_copy(obuf, out_hbm, osems, start, TAIL, slot).start()
        _out_copy(obuf, out_hbm, osems, start - TAIL, TAIL, nslot).wait()
        _out_copy(obuf, out_hbm, osems, start, TAIL, slot).wait()


@jax.jit
def kernel(x, adj, W, b):
    b2 = b.reshape(1, O)
    out = pl.pallas_call(
        _manual_kernel,
        grid=(NSTEPS,),
        in_specs=[
            pl.BlockSpec((V, C), lambda i: (0, 0)),
            pl.BlockSpec((O, C), lambda i: (0, 0)),
            pl.BlockSpec((1, O), lambda i: (0, 0)),
            pl.BlockSpec(memory_space=pl.ANY),
        ],
        out_specs=pl.BlockSpec(memory_space=pl.ANY),
        out_shape=jax.ShapeDtypeStruct((V, O), jnp.float32),
        scratch_shapes=[
            pltpu.VMEM((V, O), jnp.float32),
            pltpu.VMEM((2, MID, V), jnp.float32),
            pltpu.VMEM((2, MID, O), jnp.float32),
            pltpu.SemaphoreType.DMA((2,)),
            pltpu.SemaphoreType.DMA((2,)),
        ],
        compiler_params=pltpu.CompilerParams(
            dimension_semantics=("arbitrary",),
        ),
    )(x, W, b2, adj)
    return out


# speedup vs baseline: 1.0056x; 1.0056x over previous
import jax
import jax.numpy as jnp
from jax.experimental import pallas as pl
from jax.experimental.pallas import tpu as pltpu


def _fused_kernel(x_ref, w_ref, b_ref, adj_ref, out_ref, h_ref):
    @pl.when(pl.program_id(0) == 0)
    def _():
        h = jax.lax.dot_general(
            x_ref[...], w_ref[...],
            dimension_numbers=(((1,), (1,)), ((), ())),
            preferred_element_type=jnp.float32,
        )
        h_ref[...] = (h + b_ref[...]).astype(jnp.float32)

    p = jax.lax.dot_general(
        adj_ref[...], h_ref[...],
        dimension_numbers=(((1,), (0,)), ((), ())),
        precision=jax.lax.Precision('bfloat16'),
        preferred_element_type=jnp.float32,
    )
    out_ref[...] = p


@jax.jit
def kernel(x, adj, W, b):
    V, C = x.shape
    O = W.shape[0]
    b2 = b.reshape(1, O)

    BM = 400
    grid = (V // BM,)
    out = pl.pallas_call(
        _fused_kernel,
        grid=grid,
        in_specs=[
            pl.BlockSpec((V, C), lambda m: (0, 0),
                         pipeline_mode=pl.Buffered(buffer_count=1)),
            pl.BlockSpec((O, C), lambda m: (0, 0),
                         pipeline_mode=pl.Buffered(buffer_count=1)),
            pl.BlockSpec((1, O), lambda m: (0, 0),
                         pipeline_mode=pl.Buffered(buffer_count=1)),
            pl.BlockSpec((BM, V), lambda m: (m, 0)),
        ],
        out_specs=pl.BlockSpec((BM, O), lambda m: (m, 0)),
        out_shape=jax.ShapeDtypeStruct((V, O), jnp.float32),
        scratch_shapes=[pltpu.VMEM((V, O), jnp.float32)],
        compiler_params=pltpu.CompilerParams(
            dimension_semantics=("arbitrary",),
        ),
    )(x, W, b2, adj)
    return out


# speedup vs baseline: 1.0077x; 1.0021x over previous
import jax
import jax.numpy as jnp
from jax.experimental import pallas as pl
from jax.experimental.pallas import tpu as pltpu


def _fused_kernel(x_ref, w_ref, b_ref, adj_ref, out_ref, h_ref):
    @pl.when(pl.program_id(0) == 0)
    def _():
        h = jax.lax.dot_general(
            x_ref[...], w_ref[...],
            dimension_numbers=(((1,), (1,)), ((), ())),
            preferred_element_type=jnp.float32,
        )
        h_ref[...] = (h + b_ref[...]).astype(jnp.float32)

    p = jax.lax.dot_general(
        adj_ref[...], h_ref[...],
        dimension_numbers=(((1,), (0,)), ((), ())),
        precision=jax.lax.Precision('bfloat16'),
        preferred_element_type=jnp.float32,
    )
    out_ref[...] = p


@jax.jit
def kernel(x, adj, W, b):
    V, C = x.shape
    O = W.shape[0]
    b2 = b.reshape(1, O)

    BM = 400
    grid = (V // BM,)
    out = pl.pallas_call(
        _fused_kernel,
        grid=grid,
        in_specs=[
            pl.BlockSpec((V, C), lambda m: (0, 0)),
            pl.BlockSpec((O, C), lambda m: (0, 0)),
            pl.BlockSpec((1, O), lambda m: (0, 0)),
            pl.BlockSpec((BM, V), lambda m: (m, 0)),
        ],
        out_specs=pl.BlockSpec((BM, O), lambda m: (m, 0)),
        out_shape=jax.ShapeDtypeStruct((V, O), jnp.float32),
        scratch_shapes=[pltpu.VMEM((V, O), jnp.float32)],
        compiler_params=pltpu.CompilerParams(
            dimension_semantics=("arbitrary",),
        ),
    )(x, W, b2, adj)
    return out
